# Initial kernel scaffold; baseline (speedup 1.0000x reference)
import jax
import jax.numpy as jnp
from jax.experimental import pallas as pl


def kernel(hidden_states, ln1_0, wq_0, wk_0, wv_0, wo_0, ln2_0, wi_0, wf_0, ln1_1, wq_1, wk_1, wv_1, wo_1, ln2_1, wi_1, wf_1, gate_1, e1_1, e2_1, rel_bias, final_ln):
    raise NotImplementedError("write your pallas kernel here")


# speedup vs baseline: 23.0305x; 23.0305x over previous
import functools
import numpy as np
import jax
import jax.numpy as jnp
from jax.experimental import pallas as pl
from jax.experimental.pallas import tpu as pltpu

S = 2048
D = 768
NH = 12
DK = 64
DKP = 128
FF = 2048
NE = 8
NB = 32
MD = 128
EPS = 1e-6
_EXACT = jax.lax.Precision.HIGHEST


def _bdot(a, b, dims=None):
    ab = a.astype(jnp.bfloat16)
    bb = b.astype(jnp.bfloat16)
    if dims is None:
        dims = (((a.ndim - 1,), (0,)), ((), ()))
    return jax.lax.dot_general(ab, bb, dims,
                               preferred_element_type=jnp.float32)

BQ = 256
BS = 256
NQB = S // BQ
NSB = S // BS
AW = 2 * S
DIAGW = AW + BQ


def _ln_vec(x, w):
    var = jnp.mean(jnp.square(x), axis=-1, keepdims=True)
    return x * jax.lax.rsqrt(var + EPS) * w


def _diag_body(rb_ref, out_ref):
    pos = jax.lax.broadcasted_iota(jnp.int32, (1, DIAGW), 1)
    d = pos - (BQ + S - 1)
    nbh = NB // 2
    me = nbh // 2
    ret = (d > 0).astype(jnp.int32) * nbh
    n = jnp.abs(d)
    vl = me + (jnp.log(n.astype(jnp.float32) / me + 1e-9)
               / np.log(MD / me) * (nbh - me)).astype(jnp.int32)
    vl = jnp.minimum(vl, nbh - 1)
    bucket = ret + jnp.where(n < me, n, vl)
    j = jax.lax.broadcasted_iota(jnp.int32, (NB, DIAGW), 0)
    oh = (bucket == j).astype(jnp.float32)
    out_ref[...] = jax.lax.dot_general(
        rb_ref[...], oh, (((0,), (0,)), ((), ())),
        preferred_element_type=jnp.float32, precision=_EXACT)


def _diag_call(rel_bias):
    return pl.pallas_call(
        _diag_body,
        out_shape=jax.ShapeDtypeStruct((NH, DIAGW), jnp.float32),
    )(rel_bias)


def _qkv_body(x_ref, lnw_ref, w_ref, out_ref):
    nx = _ln_vec(x_ref[...], lnw_ref[...])
    out_ref[...] = _bdot(nx, w_ref[...])


def _qkv_call(h, lnw, wqkv):
    n = wqkv.shape[1]
    return pl.pallas_call(
        _qkv_body,
        grid=(NSB,),
        in_specs=[
            pl.BlockSpec((BS, D), lambda i: (i, 0)),
            pl.BlockSpec((1, D), lambda i: (0, 0)),
            pl.BlockSpec((D, n), lambda i: (0, 0)),
        ],
        out_specs=pl.BlockSpec((BS, n), lambda i: (i, 0)),
        out_shape=jax.ShapeDtypeStruct((S, n), jnp.float32),
    )(h, lnw.reshape(1, D), wqkv)


def _attn_body(diag_ref, q_ref, k_ref, v_ref, o_ref, a_ref):
    qb = pl.program_id(1)

    @pl.when(qb == 0)
    def _build():
        for r in range(BQ):
            a_ref[r, :] = diag_ref[0, 0, BQ - 1 - r:BQ - 1 - r + AW]

    q = q_ref[...]
    k = k_ref[...]
    scores = _bdot(q, k, (((1,), (1,)), ((), ())))
    scores = scores + a_ref[:, pl.ds(S - BQ * qb, S)]
    m = jnp.max(scores, axis=1, keepdims=True)
    p = jnp.exp(scores - m)
    den = jnp.sum(p, axis=1, keepdims=True)
    o = _bdot(p, v_ref[...], (((1,), (0,)), ((), ())))
    o_ref[...] = o / den


def _attn_call(qkv, diagk3):
    return pl.pallas_call(
        _attn_body,
        grid=(NH, NQB),
        in_specs=[
            pl.BlockSpec((1, 1, DIAGW), lambda h, qb: (h, 0, 0)),
            pl.BlockSpec((BQ, DKP), lambda h, qb: (qb, h)),
            pl.BlockSpec((S, DKP), lambda h, qb: (0, NH + h)),
            pl.BlockSpec((S, DKP), lambda h, qb: (0, 2 * NH + h)),
        ],
        out_specs=pl.BlockSpec((BQ, DKP), lambda h, qb: (qb, h)),
        out_shape=jax.ShapeDtypeStruct((S, NH * DKP), jnp.float32),
        scratch_shapes=[pltpu.VMEM((BQ, AW), jnp.float32)],
    )(diagk3, qkv, qkv, qkv)


def _oproj_body(h_ref, o_ref, wo_ref, lnw_ref, hn_ref, nx_ref):
    hn = h_ref[...] + _bdot(o_ref[...], wo_ref[...])
    hn_ref[...] = hn
    nx_ref[...] = _ln_vec(hn, lnw_ref[...])


def _oproj_call(h, o, wo_p, ln2):
    return pl.pallas_call(
        _oproj_body,
        grid=(NSB,),
        in_specs=[
            pl.BlockSpec((BS, D), lambda i: (i, 0)),
            pl.BlockSpec((BS, NH * DKP), lambda i: (i, 0)),
            pl.BlockSpec((NH * DKP, D), lambda i: (0, 0)),
            pl.BlockSpec((1, D), lambda i: (0, 0)),
        ],
        out_specs=[
            pl.BlockSpec((BS, D), lambda i: (i, 0)),
            pl.BlockSpec((BS, D), lambda i: (i, 0)),
        ],
        out_shape=[
            jax.ShapeDtypeStruct((S, D), jnp.float32),
            jax.ShapeDtypeStruct((S, D), jnp.float32),
        ],
    )(h, o, wo_p, ln2.reshape(1, D))


def _ffn_body(nx_ref, hn_ref, wi_ref, wf_ref, out_ref):
    t = jnp.maximum(_bdot(nx_ref[...], wi_ref[...]), 0.0)
    y = _bdot(t, wf_ref[...])
    out_ref[...] = hn_ref[...] + y


def _ffn_call(nx2, hn, wi, wf):
    return pl.pallas_call(
        _ffn_body,
        grid=(NSB,),
        in_specs=[
            pl.BlockSpec((BS, D), lambda i: (i, 0)),
            pl.BlockSpec((BS, D), lambda i: (i, 0)),
            pl.BlockSpec((D, FF), lambda i: (0, 0)),
            pl.BlockSpec((FF, D), lambda i: (0, 0)),
        ],
        out_specs=pl.BlockSpec((BS, D), lambda i: (i, 0)),
        out_shape=jax.ShapeDtypeStruct((S, D), jnp.float32),
    )(nx2, hn, wi, wf)


def _route_body(x_ref, g_ref, w_ref):
    logits = _bdot(x_ref[...], g_ref[...])
    iota = jax.lax.broadcasted_iota(jnp.int32, (S, NE), 1)
    m1 = jnp.max(logits, axis=1, keepdims=True)
    i1 = jnp.min(jnp.where(logits == m1, iota, NE), axis=1, keepdims=True)
    masked = jnp.where(iota == i1, -1e30, logits)
    m2 = jnp.max(masked, axis=1, keepdims=True)
    i2 = jnp.min(jnp.where(masked == m2, iota, NE), axis=1, keepdims=True)
    g2 = 1.0 / (1.0 + jnp.exp(m1 - m2))
    g1 = 1.0 - g2
    w_ref[...] = (jnp.where(iota == i1, g1, 0.0)
                  + jnp.where(iota == i2, g2, 0.0))


def _route_call(flat, gate):
    return pl.pallas_call(
        _route_body,
        out_shape=jax.ShapeDtypeStruct((S, NE), jnp.float32),
    )(flat, gate)


def _moe_body(acc_ref, x_ref, w_ref, e1_ref, e2_ref, out_ref):
    e = pl.program_id(0)
    x = x_ref[...]
    t = jax.nn.gelu(_bdot(x, e1_ref[0]))
    y = _bdot(t, e2_ref[0])
    iota = jax.lax.broadcasted_iota(jnp.int32, (BS, NE), 1)
    w = jnp.sum(jnp.where(iota == e, w_ref[...], 0.0), axis=1, keepdims=True)
    out_ref[...] = acc_ref[...] + w * y


def _moe_call(acc, flat, wts, e1, e2):
    return pl.pallas_call(
        _moe_body,
        grid=(NE, NSB),
        in_specs=[
            pl.BlockSpec((BS, D), lambda e, i: (i, 0)),
            pl.BlockSpec((BS, D), lambda e, i: (i, 0)),
            pl.BlockSpec((BS, NE), lambda e, i: (i, 0)),
            pl.BlockSpec((1, D, FF), lambda e, i: (e, 0, 0)),
            pl.BlockSpec((1, FF, D), lambda e, i: (e, 0, 0)),
        ],
        out_specs=pl.BlockSpec((BS, D), lambda e, i: (i, 0)),
        out_shape=jax.ShapeDtypeStruct((S, D), jnp.float32),
        input_output_aliases={0: 0},
    )(acc, flat, wts, e1, e2)


def _finalln_body(x_ref, w_ref, out_ref):
    out_ref[...] = _ln_vec(x_ref[...], w_ref[...])


def _finalln_call(h, w):
    return pl.pallas_call(
        _finalln_body,
        grid=(NSB,),
        in_specs=[
            pl.BlockSpec((BS, D), lambda i: (i, 0)),
            pl.BlockSpec((1, D), lambda i: (0, 0)),
        ],
        out_specs=pl.BlockSpec((BS, D), lambda i: (i, 0)),
        out_shape=jax.ShapeDtypeStruct((S, D), jnp.float32),
    )(h, w.reshape(1, D))


def _pad_heads_cols(w):
    w3 = w.reshape(D, NH, DK)
    w3 = jnp.pad(w3, ((0, 0), (0, 0), (0, DKP - DK)))
    return w3.reshape(D, NH * DKP)


def _pad_heads_rows(w):
    w3 = w.reshape(NH, DK, D)
    w3 = jnp.pad(w3, ((0, 0), (0, DKP - DK), (0, 0)))
    return w3.reshape(NH * DKP, D)


def kernel(hidden_states, ln1_0, wq_0, wk_0, wv_0, wo_0, ln2_0, wi_0, wf_0,
           ln1_1, wq_1, wk_1, wv_1, wo_1, ln2_1, wi_1, wf_1,
           gate_1, e1_1, e2_1, rel_bias, final_ln):
    h = hidden_states.reshape(S, D)
    diagk3 = _diag_call(rel_bias).reshape(NH, 1, DIAGW)
    layers = [
        (ln1_0, wq_0, wk_0, wv_0, wo_0, ln2_0, wi_0, wf_0, False),
        (ln1_1, wq_1, wk_1, wv_1, wo_1, ln2_1, wi_1, wf_1, True),
    ]
    for ln1, wq, wk, wv, wo, ln2, wi, wf, is_moe in layers:
        wqkv = jnp.concatenate(
            [_pad_heads_cols(wq), _pad_heads_cols(wk), _pad_heads_cols(wv)],
            axis=1)
        qkv = _qkv_call(h, ln1, wqkv)
        o = _attn_call(qkv, diagk3)
        h, nx2 = _oproj_call(h, o, _pad_heads_rows(wo), ln2)
        y = _ffn_call(nx2, h, wi, wf)
        if is_moe:
            wts = _route_call(nx2, gate_1)
            y = _moe_call(y, nx2, wts, e1_1, e2_1)
        h = y
    out = _finalln_call(h, final_ln)
    return out.reshape(1, S, D)
